# baseline (speedup 1.0000x reference)
import functools

import jax
import jax.numpy as jnp
from jax import lax
from jax.experimental import pallas as pl
from jax.experimental.pallas import tpu as pltpu
from jax.experimental.pallas import tpu_sc as plsc

H, W = 200, 272
C = 256
M = 14
N_ROIS = 512
P = N_ROIS * M * M
HW = H * W
SCALE = 0.25

NW = 32
PPW = P // NW
CHUNK = 32
NCHUNK = PPW // CHUNK

T_STEPS = 25
T_COLS = HW // T_STEPS

NB = 32
MM_ROWS = 1024
MM_STEPS = P // MM_ROWS


def _transpose_body(src_ref, dst_ref):
    dst_ref[...] = src_ref[...].T


def _prep_body(mp_ref, idx_ref, wtb_ref):
    b = mp_ref[...] * SCALE
    x1, y1, x2, y2 = b[:, 0:1], b[:, 1:2], b[:, 2:3], b[:, 3:4]
    bin_w = jnp.maximum(x2 - x1, 1.0) / M
    bin_h = jnp.maximum(y2 - y1, 1.0) / M
    g = lax.broadcasted_iota(jnp.int32, (1, M), 1).astype(jnp.float32) + 0.5
    x = jnp.clip(x1 + g * bin_w, 0.0, W - 1.0)
    y = jnp.clip(y1 + g * bin_h, 0.0, H - 1.0)
    x0f = jnp.floor(x)
    y0f = jnp.floor(y)
    x0 = x0f.astype(jnp.int32)
    y0 = y0f.astype(jnp.int32)
    lx = x - x0f
    ly = y - y0f
    hx = 1.0 - lx
    hy = 1.0 - ly
    row0 = y0 * W
    i00 = row0[:, :, None] + x0[:, None, :]
    i10 = i00 + W
    cap = HW - 1
    idx_ref[:, :, :, 0] = i00
    idx_ref[:, :, :, 1] = jnp.minimum(i00 + 1, cap)
    idx_ref[:, :, :, 2] = jnp.minimum(i10, cap)
    idx_ref[:, :, :, 3] = jnp.minimum(i10 + 1, cap)
    w00 = hy[:, :, None] * hx[:, None, :]
    w01 = hy[:, :, None] * lx[:, None, :]
    w10 = ly[:, :, None] * hx[:, None, :]
    w11 = ly[:, :, None] * lx[:, None, :]
    wtb_ref[:, :, :, 0:16] = jnp.broadcast_to(w00[..., None], (NB, M, M, 16))
    wtb_ref[:, :, :, 16:32] = jnp.broadcast_to(w01[..., None], (NB, M, M, 16))
    wtb_ref[:, :, :, 32:48] = jnp.broadcast_to(w10[..., None], (NB, M, M, 16))
    wtb_ref[:, :, :, 48:64] = jnp.broadcast_to(w11[..., None], (NB, M, M, 16))


def _sc_gather_body(table, idxc, wtb, out,
                    ia, ib, va, vb, wva, wvb, oa, ob,
                    sga, sgb, swa, swb):
    wid = lax.axis_index("s") * 2 + lax.axis_index("c")
    base = wid * PPW

    slots = ((ia, va, wva, oa, sga, swa),
             (ib, vb, wvb, ob, sgb, swb))

    def fire(ci, s):
        i_s, v_s, wv_s, _, sg_s, _ = s
        p0 = base + ci * CHUNK
        pltpu.sync_copy(idxc.at[pl.ds(4 * p0, 4 * CHUNK)], i_s)
        pltpu.async_copy(table.at[i_s], v_s, sg_s)
        pltpu.async_copy(wtb.at[pl.ds(p0, CHUNK)], wv_s, sg_s)

    def drain_g(s):
        i_s, v_s, wv_s, _, sg_s, _ = s
        pltpu.make_async_copy(table.at[i_s], v_s, sg_s).wait()
        pltpu.make_async_copy(wtb.at[pl.ds(0, CHUNK)], wv_s, sg_s).wait()

    def fire_w(ci, s):
        _, _, _, o_s, _, sw_s = s
        p0 = base + ci * CHUNK
        pltpu.async_copy(o_s, out.at[pl.ds(p0, CHUNK)], sw_s)

    def drain_w(s):
        _, _, _, o_s, _, sw_s = s
        pltpu.make_async_copy(o_s, out.at[pl.ds(0, CHUNK)], sw_s).wait()

    def combine(s):
        _, v_s, wv_s, o_s, _, _ = s

        def point(p, pc):
            w0 = wv_s[p, pl.ds(0, 16)]
            w1 = wv_s[p, pl.ds(16, 16)]
            w2 = wv_s[p, pl.ds(32, 16)]
            w3 = wv_s[p, pl.ds(48, 16)]
            q = 4 * p
            for k in range(C // 16):
                sl = pl.ds(k * 16, 16)
                r = (w0 * v_s[q, sl] + w1 * v_s[q + 1, sl]
                     + w2 * v_s[q + 2, sl] + w3 * v_s[q + 3, sl])
                o_s[p, sl] = r
            return pc

        lax.fori_loop(0, CHUNK, point, 0)

    fire(0, slots[0])
    fire(1, slots[1])

    for b in range(2):
        s = slots[b]
        drain_g(s)
        combine(s)
        fire_w(b, s)
        fire(b + 2, s)

    def pair(g, carry):
        for b in range(2):
            ci = 2 * g + b
            s = slots[b]
            drain_g(s)
            drain_w(s)
            combine(s)
            fire_w(ci, s)
            fire(ci + 2, s)
        return carry

    lax.fori_loop(1, NCHUNK // 2 - 1, pair, 0)

    for b in range(2):
        s = slots[b]
        drain_g(s)
        drain_w(s)
        combine(s)
        fire_w(NCHUNK - 2 + b, s)

    drain_w(slots[0])
    drain_w(slots[1])


@functools.lru_cache(maxsize=1)
def _sc_gather():
    return pl.kernel(
        _sc_gather_body,
        mesh=plsc.VectorSubcoreMesh(core_axis_name="c", subcore_axis_name="s"),
        out_type=jax.ShapeDtypeStruct((P, C), jnp.float32),
        scratch_types=[
            pltpu.VMEM((4 * CHUNK,), jnp.int32),
            pltpu.VMEM((4 * CHUNK,), jnp.int32),
            pltpu.VMEM((4 * CHUNK, C), jnp.float32),
            pltpu.VMEM((4 * CHUNK, C), jnp.float32),
            pltpu.VMEM((CHUNK, 64), jnp.float32),
            pltpu.VMEM((CHUNK, 64), jnp.float32),
            pltpu.VMEM((CHUNK, C), jnp.float32),
            pltpu.VMEM((CHUNK, C), jnp.float32),
            pltpu.SemaphoreType.DMA,
            pltpu.SemaphoreType.DMA,
            pltpu.SemaphoreType.DMA,
            pltpu.SemaphoreType.DMA,
        ],
    )


def _mm_body(x_ref, wh_ref, bh_ref, w2_ref, b2_ref, wp_ref, bp_ref, o_ref):
    x = x_ref[...]
    h = jnp.maximum(
        jnp.dot(x, wh_ref[...], preferred_element_type=jnp.float32)
        + bh_ref[...], 0.0)
    u = jnp.maximum(
        jnp.dot(h, w2_ref[...], preferred_element_type=jnp.float32)
        + b2_ref[...], 0.0)
    z = jnp.dot(u, wp_ref[...], preferred_element_type=jnp.float32) + bp_ref[...]
    o_ref[...] = jax.nn.sigmoid(z)


def kernel(features, mask_proposals, w_head, b_head, w_deconv, b_deconv,
           w_pred, b_pred):
    f2 = features.reshape(C, HW)
    table = pl.pallas_call(
        _transpose_body,
        grid=(T_STEPS,),
        in_specs=[pl.BlockSpec((C, T_COLS), lambda i: (0, i))],
        out_specs=pl.BlockSpec((T_COLS, C), lambda i: (i, 0)),
        out_shape=jax.ShapeDtypeStruct((HW, C), jnp.float32),
    )(f2)

    idxc, wtb = pl.pallas_call(
        _prep_body,
        grid=(N_ROIS // NB,),
        in_specs=[pl.BlockSpec((NB, 4), lambda i: (i, 0))],
        out_specs=[pl.BlockSpec((NB, M, M, 4), lambda i: (i, 0, 0, 0)),
                   pl.BlockSpec((NB, M, M, 64), lambda i: (i, 0, 0, 0))],
        out_shape=[jax.ShapeDtypeStruct((N_ROIS, M, M, 4), jnp.int32),
                   jax.ShapeDtypeStruct((N_ROIS, M, M, 64), jnp.float32)],
    )(mask_proposals)
    idxc = idxc.reshape(4 * P)
    wtb = wtb.reshape(P, 64)

    roi = _sc_gather()(table, idxc, wtb)

    whT = jnp.transpose(w_head)
    bh = b_head.reshape(1, 128)
    w2r = w_deconv.reshape(128, 4 * 64)
    b2 = jnp.repeat(b_deconv, 4).reshape(1, 256)
    wp4 = (w_pred[0].reshape(64, 1, 1)
           * jnp.eye(4, dtype=w_pred.dtype).reshape(1, 4, 4)).reshape(256, 4)
    bp = jnp.broadcast_to(b_pred[0:1], (1, 4))

    val = pl.pallas_call(
        _mm_body,
        grid=(MM_STEPS,),
        in_specs=[
            pl.BlockSpec((MM_ROWS, C), lambda i: (i, 0)),
            pl.BlockSpec((C, 128), lambda i: (0, 0)),
            pl.BlockSpec((1, 128), lambda i: (0, 0)),
            pl.BlockSpec((128, 256), lambda i: (0, 0)),
            pl.BlockSpec((1, 256), lambda i: (0, 0)),
            pl.BlockSpec((256, 4), lambda i: (0, 0)),
            pl.BlockSpec((1, 4), lambda i: (0, 0)),
        ],
        out_specs=pl.BlockSpec((MM_ROWS, 4), lambda i: (i, 0)),
        out_shape=jax.ShapeDtypeStruct((P, 4), jnp.float32),
    )(roi, whT, bh, w2r, b2, wp4, bp)

    out = val.reshape(N_ROIS, M, M, 2, 2).transpose(0, 1, 3, 2, 4)
    return out.reshape(N_ROIS, 2 * M, 2 * M)


# speedup vs baseline: 1.3167x; 1.3167x over previous
import functools

import jax
import jax.numpy as jnp
from jax import lax
from jax.experimental import pallas as pl
from jax.experimental.pallas import tpu as pltpu
from jax.experimental.pallas import tpu_sc as plsc

H, W = 200, 272
C = 256
M = 14
N_ROIS = 512
P = N_ROIS * M * M
HW = H * W
SCALE = 0.25

NW = 32
PPW = P // NW
CHUNK = 32
NCHUNK = PPW // CHUNK

T_STEPS = 25
T_COLS = HW // T_STEPS

NB = 32
MM_ROWS = 1024
MM_STEPS = P // MM_ROWS


def _transpose_body(src_ref, dst_ref):
    dst_ref[...] = src_ref[...].T


def _prep_body(mp_ref, i00_ref, i01_ref, i10_ref, i11_ref, wtb_ref):
    b = mp_ref[...] * SCALE
    x1, y1, x2, y2 = b[:, 0:1], b[:, 1:2], b[:, 2:3], b[:, 3:4]
    bin_w = jnp.maximum(x2 - x1, 1.0) / M
    bin_h = jnp.maximum(y2 - y1, 1.0) / M
    g = lax.broadcasted_iota(jnp.int32, (1, M), 1).astype(jnp.float32) + 0.5
    x = jnp.clip(x1 + g * bin_w, 0.0, W - 1.0)
    y = jnp.clip(y1 + g * bin_h, 0.0, H - 1.0)
    x0f = jnp.floor(x)
    y0f = jnp.floor(y)
    x0 = x0f.astype(jnp.int32)
    y0 = y0f.astype(jnp.int32)
    lx = x - x0f
    ly = y - y0f
    hx = 1.0 - lx
    hy = 1.0 - ly
    row0 = y0 * W
    i00 = row0[:, :, None] + x0[:, None, :]
    i10 = i00 + W
    cap = HW - 1
    i00_ref[...] = i00
    i01_ref[...] = jnp.minimum(i00 + 1, cap)
    i10_ref[...] = jnp.minimum(i10, cap)
    i11_ref[...] = jnp.minimum(i10 + 1, cap)
    w00 = hy[:, :, None] * hx[:, None, :]
    w01 = hy[:, :, None] * lx[:, None, :]
    w10 = ly[:, :, None] * hx[:, None, :]
    w11 = ly[:, :, None] * lx[:, None, :]
    wtb_ref[:, :, :, 0:16] = jnp.broadcast_to(w00[..., None], (NB, M, M, 16))
    wtb_ref[:, :, :, 16:32] = jnp.broadcast_to(w01[..., None], (NB, M, M, 16))
    wtb_ref[:, :, :, 32:48] = jnp.broadcast_to(w10[..., None], (NB, M, M, 16))
    wtb_ref[:, :, :, 48:64] = jnp.broadcast_to(w11[..., None], (NB, M, M, 16))


def _sc_gather_body(table, x00, x01, x10, x11, wtb, out,
                    i0a, i1a, i2a, i3a, v0a, v1a, v2a, v3a, wva, oa,
                    i0b, i1b, i2b, i3b, v0b, v1b, v2b, v3b, wvb, ob,
                    sga, sgb, swa, swb):
    wid = lax.axis_index("s") * 2 + lax.axis_index("c")
    base = wid * PPW

    slots = (((i0a, i1a, i2a, i3a), (v0a, v1a, v2a, v3a), wva, oa, sga, swa),
             ((i0b, i1b, i2b, i3b), (v0b, v1b, v2b, v3b), wvb, ob, sgb, swb))
    srcs = (x00, x01, x10, x11)

    def fire(ci, s):
        idx, vbufs, wv_s, _, sg_s, _ = s
        p0 = base + ci * CHUNK
        for j in range(4):
            pltpu.sync_copy(srcs[j].at[pl.ds(p0, CHUNK)], idx[j])
        for j in range(4):
            pltpu.async_copy(table.at[idx[j]], vbufs[j], sg_s)
        pltpu.async_copy(wtb.at[pl.ds(p0, CHUNK)], wv_s, sg_s)

    def drain_g(s):
        idx, vbufs, wv_s, _, sg_s, _ = s
        for j in range(4):
            pltpu.make_async_copy(table.at[idx[j]], vbufs[j], sg_s).wait()
        pltpu.make_async_copy(wtb.at[pl.ds(0, CHUNK)], wv_s, sg_s).wait()

    def fire_w(ci, s):
        _, _, _, o_s, _, sw_s = s
        p0 = base + ci * CHUNK
        pltpu.async_copy(o_s, out.at[pl.ds(p0, CHUNK)], sw_s)

    def drain_w(s):
        _, _, _, o_s, _, sw_s = s
        pltpu.make_async_copy(o_s, out.at[pl.ds(0, CHUNK)], sw_s).wait()

    def combine(s):
        _, (v0, v1, v2, v3), wv_s, o_s, _, _ = s

        def point(p, pc):
            w0 = wv_s[p, pl.ds(0, 16)]
            w1 = wv_s[p, pl.ds(16, 16)]
            w2 = wv_s[p, pl.ds(32, 16)]
            w3 = wv_s[p, pl.ds(48, 16)]
            for k in range(C // 16):
                sl = pl.ds(k * 16, 16)
                r = (w0 * v0[p, sl] + w1 * v1[p, sl]
                     + w2 * v2[p, sl] + w3 * v3[p, sl])
                o_s[p, sl] = r
            return pc

        lax.fori_loop(0, CHUNK, point, 0)

    fire(0, slots[0])
    fire(1, slots[1])

    for b in range(2):
        s = slots[b]
        drain_g(s)
        combine(s)
        fire_w(b, s)
        fire(b + 2, s)

    def pair(g, carry):
        for b in range(2):
            ci = 2 * g + b
            s = slots[b]
            drain_g(s)
            drain_w(s)
            combine(s)
            fire_w(ci, s)
            fire(ci + 2, s)
        return carry

    lax.fori_loop(1, NCHUNK // 2 - 1, pair, 0)

    for b in range(2):
        s = slots[b]
        drain_g(s)
        drain_w(s)
        combine(s)
        fire_w(NCHUNK - 2 + b, s)

    drain_w(slots[0])
    drain_w(slots[1])


@functools.lru_cache(maxsize=1)
def _sc_gather():
    ibuf = pltpu.VMEM((CHUNK,), jnp.int32)
    vbuf = pltpu.VMEM((CHUNK, C), jnp.float32)
    wbuf = pltpu.VMEM((CHUNK, 64), jnp.float32)
    return pl.kernel(
        _sc_gather_body,
        mesh=plsc.VectorSubcoreMesh(core_axis_name="c", subcore_axis_name="s"),
        out_type=jax.ShapeDtypeStruct((P, C), jnp.float32),
        scratch_types=[
            ibuf, ibuf, ibuf, ibuf, vbuf, vbuf, vbuf, vbuf, wbuf, vbuf,
            ibuf, ibuf, ibuf, ibuf, vbuf, vbuf, vbuf, vbuf, wbuf, vbuf,
            pltpu.SemaphoreType.DMA,
            pltpu.SemaphoreType.DMA,
            pltpu.SemaphoreType.DMA,
            pltpu.SemaphoreType.DMA,
        ],
    )


def _mm_body(x_ref, wh_ref, bh_ref, w2_ref, b2_ref, wp_ref, bp_ref, o_ref):
    x = x_ref[...]
    h = jnp.maximum(
        jnp.dot(x, wh_ref[...], preferred_element_type=jnp.float32)
        + bh_ref[...], 0.0)
    u = jnp.maximum(
        jnp.dot(h, w2_ref[...], preferred_element_type=jnp.float32)
        + b2_ref[...], 0.0)
    z = jnp.dot(u, wp_ref[...], preferred_element_type=jnp.float32) + bp_ref[...]
    o_ref[...] = jax.nn.sigmoid(z)


def kernel(features, mask_proposals, w_head, b_head, w_deconv, b_deconv,
           w_pred, b_pred):
    f2 = features.reshape(C, HW)
    table = pl.pallas_call(
        _transpose_body,
        grid=(T_STEPS,),
        in_specs=[pl.BlockSpec((C, T_COLS), lambda i: (0, i))],
        out_specs=pl.BlockSpec((T_COLS, C), lambda i: (i, 0)),
        out_shape=jax.ShapeDtypeStruct((HW, C), jnp.float32),
    )(f2)

    ispec = pl.BlockSpec((NB, M, M), lambda i: (i, 0, 0))
    ishape = jax.ShapeDtypeStruct((N_ROIS, M, M), jnp.int32)
    x00, x01, x10, x11, wtb = pl.pallas_call(
        _prep_body,
        grid=(N_ROIS // NB,),
        in_specs=[pl.BlockSpec((NB, 4), lambda i: (i, 0))],
        out_specs=[ispec, ispec, ispec, ispec,
                   pl.BlockSpec((NB, M, M, 64), lambda i: (i, 0, 0, 0))],
        out_shape=[ishape, ishape, ishape, ishape,
                   jax.ShapeDtypeStruct((N_ROIS, M, M, 64), jnp.float32)],
    )(mask_proposals)
    x00 = x00.reshape(P)
    x01 = x01.reshape(P)
    x10 = x10.reshape(P)
    x11 = x11.reshape(P)
    wtb = wtb.reshape(P, 64)

    roi = _sc_gather()(table, x00, x01, x10, x11, wtb)

    whT = jnp.transpose(w_head)
    bh = b_head.reshape(1, 128)
    w2r = w_deconv.reshape(128, 4 * 64)
    b2 = jnp.repeat(b_deconv, 4).reshape(1, 256)
    wp4 = (w_pred[0].reshape(64, 1, 1)
           * jnp.eye(4, dtype=w_pred.dtype).reshape(1, 4, 4)).reshape(256, 4)
    bp = jnp.broadcast_to(b_pred[0:1], (1, 4))

    val = pl.pallas_call(
        _mm_body,
        grid=(MM_STEPS,),
        in_specs=[
            pl.BlockSpec((MM_ROWS, C), lambda i: (i, 0)),
            pl.BlockSpec((C, 128), lambda i: (0, 0)),
            pl.BlockSpec((1, 128), lambda i: (0, 0)),
            pl.BlockSpec((128, 256), lambda i: (0, 0)),
            pl.BlockSpec((1, 256), lambda i: (0, 0)),
            pl.BlockSpec((256, 4), lambda i: (0, 0)),
            pl.BlockSpec((1, 4), lambda i: (0, 0)),
        ],
        out_specs=pl.BlockSpec((MM_ROWS, 4), lambda i: (i, 0)),
        out_shape=jax.ShapeDtypeStruct((P, 4), jnp.float32),
    )(roi, whT, bh, w2r, b2, wp4, bp)

    out = val.reshape(N_ROIS, M, M, 2, 2).transpose(0, 1, 3, 2, 4)
    return out.reshape(N_ROIS, 2 * M, 2 * M)


# speedup vs baseline: 1.4420x; 1.0952x over previous
import functools

import jax
import jax.numpy as jnp
from jax import lax
from jax.experimental import pallas as pl
from jax.experimental.pallas import tpu as pltpu
from jax.experimental.pallas import tpu_sc as plsc

H, W = 200, 272
C = 256
M = 14
N_ROIS = 512
P = N_ROIS * M * M
HW = H * W
SCALE = 0.25

NW = 32
PPW = P // NW
CHUNK = 32
NCHUNK = PPW // CHUNK

T_STEPS = 25
T_COLS = HW // T_STEPS

NB = 32
MM_ROWS = 1024
MM_STEPS = P // MM_ROWS


def _transpose_body(src_ref, dst_ref):
    dst_ref[...] = src_ref[...].reshape(C, T_COLS).T


def _prep_body(mp_ref, i00_ref, i01_ref, i10_ref, i11_ref, wtb_ref):
    b = mp_ref[...] * SCALE
    x1, y1, x2, y2 = b[:, 0:1], b[:, 1:2], b[:, 2:3], b[:, 3:4]
    bin_w = jnp.maximum(x2 - x1, 1.0) / M
    bin_h = jnp.maximum(y2 - y1, 1.0) / M
    g = lax.broadcasted_iota(jnp.int32, (1, M), 1).astype(jnp.float32) + 0.5
    x = jnp.clip(x1 + g * bin_w, 0.0, W - 1.0)
    y = jnp.clip(y1 + g * bin_h, 0.0, H - 1.0)
    x0f = jnp.floor(x)
    y0f = jnp.floor(y)
    x0 = x0f.astype(jnp.int32)
    y0 = y0f.astype(jnp.int32)
    lx = x - x0f
    ly = y - y0f
    hx = 1.0 - lx
    hy = 1.0 - ly
    row0 = y0 * W
    i00 = row0[:, :, None] + x0[:, None, :]
    i10 = i00 + W
    cap = HW - 1
    i00_ref[...] = i00
    i01_ref[...] = jnp.minimum(i00 + 1, cap)
    i10_ref[...] = jnp.minimum(i10, cap)
    i11_ref[...] = jnp.minimum(i10 + 1, cap)
    w00 = hy[:, :, None] * hx[:, None, :]
    w01 = hy[:, :, None] * lx[:, None, :]
    w10 = ly[:, :, None] * hx[:, None, :]
    w11 = ly[:, :, None] * lx[:, None, :]
    wtb_ref[:, :, :, 0:16] = jnp.broadcast_to(w00[..., None], (NB, M, M, 16))
    wtb_ref[:, :, :, 16:32] = jnp.broadcast_to(w01[..., None], (NB, M, M, 16))
    wtb_ref[:, :, :, 32:48] = jnp.broadcast_to(w10[..., None], (NB, M, M, 16))
    wtb_ref[:, :, :, 48:64] = jnp.broadcast_to(w11[..., None], (NB, M, M, 16))


def _sc_gather_body(table, x00, x01, x10, x11, wtb, out,
                    i0a, i1a, i2a, i3a, v0a, v1a, v2a, v3a, wva, oa,
                    i0b, i1b, i2b, i3b, v0b, v1b, v2b, v3b, wvb, ob,
                    sga, sgb, swa, swb):
    wid = lax.axis_index("s") * 2 + lax.axis_index("c")
    base = wid * PPW

    slots = (((i0a, i1a, i2a, i3a), (v0a, v1a, v2a, v3a), wva, oa, sga, swa),
             ((i0b, i1b, i2b, i3b), (v0b, v1b, v2b, v3b), wvb, ob, sgb, swb))
    srcs = (x00, x01, x10, x11)

    def fire(ci, s):
        idx, vbufs, wv_s, _, sg_s, _ = s
        p0 = base + ci * CHUNK
        for j in range(4):
            pltpu.sync_copy(srcs[j].at[pl.ds(p0, CHUNK)], idx[j])
        for j in range(4):
            pltpu.async_copy(table.at[idx[j]], vbufs[j], sg_s)
        pltpu.async_copy(wtb.at[pl.ds(p0, CHUNK)], wv_s, sg_s)

    def drain_g(s):
        idx, vbufs, wv_s, _, sg_s, _ = s
        for j in range(4):
            pltpu.make_async_copy(table.at[idx[j]], vbufs[j], sg_s).wait()
        pltpu.make_async_copy(wtb.at[pl.ds(0, CHUNK)], wv_s, sg_s).wait()

    def fire_w(ci, s):
        _, _, _, o_s, _, sw_s = s
        p0 = base + ci * CHUNK
        pltpu.async_copy(o_s, out.at[pl.ds(p0, CHUNK)], sw_s)

    def drain_w(s):
        _, _, _, o_s, _, sw_s = s
        pltpu.make_async_copy(o_s, out.at[pl.ds(0, CHUNK)], sw_s).wait()

    def combine(s):
        _, (v0, v1, v2, v3), wv_s, o_s, _, _ = s

        def point(p, pc):
            w0 = wv_s[p, pl.ds(0, 16)]
            w1 = wv_s[p, pl.ds(16, 16)]
            w2 = wv_s[p, pl.ds(32, 16)]
            w3 = wv_s[p, pl.ds(48, 16)]
            for k in range(C // 16):
                sl = pl.ds(k * 16, 16)
                r = (w0 * v0[p, sl] + w1 * v1[p, sl]
                     + w2 * v2[p, sl] + w3 * v3[p, sl])
                o_s[p, sl] = r
            return pc

        lax.fori_loop(0, CHUNK, point, 0)

    fire(0, slots[0])
    fire(1, slots[1])

    for b in range(2):
        s = slots[b]
        drain_g(s)
        combine(s)
        fire_w(b, s)
        fire(b + 2, s)

    def pair(g, carry):
        for b in range(2):
            ci = 2 * g + b
            s = slots[b]
            drain_g(s)
            drain_w(s)
            combine(s)
            fire_w(ci, s)
            fire(ci + 2, s)
        return carry

    lax.fori_loop(1, NCHUNK // 2 - 1, pair, 0)

    for b in range(2):
        s = slots[b]
        drain_g(s)
        drain_w(s)
        combine(s)
        fire_w(NCHUNK - 2 + b, s)

    drain_w(slots[0])
    drain_w(slots[1])


@functools.lru_cache(maxsize=1)
def _sc_gather():
    ibuf = pltpu.VMEM((CHUNK,), jnp.int32)
    vbuf = pltpu.VMEM((CHUNK, C), jnp.float32)
    wbuf = pltpu.VMEM((CHUNK, 64), jnp.float32)
    return pl.kernel(
        _sc_gather_body,
        mesh=plsc.VectorSubcoreMesh(core_axis_name="c", subcore_axis_name="s"),
        out_type=jax.ShapeDtypeStruct((P, C), jnp.float32),
        scratch_types=[
            ibuf, ibuf, ibuf, ibuf, vbuf, vbuf, vbuf, vbuf, wbuf, vbuf,
            ibuf, ibuf, ibuf, ibuf, vbuf, vbuf, vbuf, vbuf, wbuf, vbuf,
            pltpu.SemaphoreType.DMA,
            pltpu.SemaphoreType.DMA,
            pltpu.SemaphoreType.DMA,
            pltpu.SemaphoreType.DMA,
        ],
    )


def _mm_body(x_ref, wh_ref, bh_ref, w2_ref, b2_ref, wp_ref, bp_ref, o_ref):
    x = x_ref[...]
    h = jnp.maximum(
        jnp.dot(x, wh_ref[...], preferred_element_type=jnp.float32)
        + bh_ref[...], 0.0)
    u = jnp.maximum(
        jnp.dot(h, w2_ref[...], preferred_element_type=jnp.float32)
        + b2_ref[...], 0.0)
    z = jnp.dot(u, wp_ref[...], preferred_element_type=jnp.float32) + bp_ref[...]
    o_ref[...] = jax.nn.sigmoid(jnp.transpose(z))


def kernel(features, mask_proposals, w_head, b_head, w_deconv, b_deconv,
           w_pred, b_pred):
    table = pl.pallas_call(
        _transpose_body,
        grid=(T_STEPS,),
        in_specs=[pl.BlockSpec((C, H // T_STEPS, W), lambda i: (0, i, 0))],
        out_specs=pl.BlockSpec((T_COLS, C), lambda i: (i, 0)),
        out_shape=jax.ShapeDtypeStruct((HW, C), jnp.float32),
    )(features.reshape(C, H, W))

    ispec = pl.BlockSpec((NB, M, M), lambda i: (i, 0, 0))
    ishape = jax.ShapeDtypeStruct((N_ROIS, M, M), jnp.int32)
    x00, x01, x10, x11, wtb = pl.pallas_call(
        _prep_body,
        grid=(N_ROIS // NB,),
        in_specs=[pl.BlockSpec((NB, 4), lambda i: (i, 0))],
        out_specs=[ispec, ispec, ispec, ispec,
                   pl.BlockSpec((NB, M, M, 64), lambda i: (i, 0, 0, 0))],
        out_shape=[ishape, ishape, ishape, ishape,
                   jax.ShapeDtypeStruct((N_ROIS, M, M, 64), jnp.float32)],
    )(mask_proposals)
    x00 = x00.reshape(P)
    x01 = x01.reshape(P)
    x10 = x10.reshape(P)
    x11 = x11.reshape(P)
    wtb = wtb.reshape(P, 64)

    roi = _sc_gather()(table, x00, x01, x10, x11, wtb)

    whT = jnp.transpose(w_head)
    bh = b_head.reshape(1, 128)
    w2r = w_deconv.reshape(128, 4 * 64)
    b2 = jnp.repeat(b_deconv, 4).reshape(1, 256)
    wp4 = (w_pred[0].reshape(64, 1, 1)
           * jnp.eye(4, dtype=w_pred.dtype).reshape(1, 4, 4)).reshape(256, 4)
    bp = jnp.broadcast_to(b_pred[0:1], (1, 4))

    val = pl.pallas_call(
        _mm_body,
        grid=(MM_STEPS,),
        in_specs=[
            pl.BlockSpec((MM_ROWS, C), lambda i: (i, 0)),
            pl.BlockSpec((C, 128), lambda i: (0, 0)),
            pl.BlockSpec((1, 128), lambda i: (0, 0)),
            pl.BlockSpec((128, 256), lambda i: (0, 0)),
            pl.BlockSpec((1, 256), lambda i: (0, 0)),
            pl.BlockSpec((256, 4), lambda i: (0, 0)),
            pl.BlockSpec((1, 4), lambda i: (0, 0)),
        ],
        out_specs=pl.BlockSpec((4, MM_ROWS), lambda i: (0, i)),
        out_shape=jax.ShapeDtypeStruct((4, P), jnp.float32),
    )(roi, whT, bh, w2r, b2, wp4, bp)

    out = val.reshape(2, 2, N_ROIS, M, M).transpose(2, 3, 0, 4, 1)
    return out.reshape(N_ROIS, 2 * M, 2 * M)


# speedup vs baseline: 1.6286x; 1.1294x over previous
import functools

import jax
import jax.numpy as jnp
from jax import lax
from jax.experimental import pallas as pl
from jax.experimental.pallas import tpu as pltpu
from jax.experimental.pallas import tpu_sc as plsc

H, W = 200, 272
C = 256
M = 14
N_ROIS = 512
P = N_ROIS * M * M
HW = H * W
SCALE = 0.25

NSLICE = 2
RS = N_ROIS // NSLICE
PS = P // NSLICE

NW = 32
PPW = PS // NW
CHUNK = 32
NCHUNK = PPW // CHUNK

T_STEPS = 25
T_COLS = HW // T_STEPS

NB = 32
MM_ROWS = 1024
MM_STEPS = PS // MM_ROWS


def _transpose_body(src_ref, dst_ref):
    dst_ref[...] = src_ref[...].reshape(C, T_COLS).T


def _prep_body(mp_ref, i00_ref, i01_ref, i10_ref, i11_ref, wtb_ref):
    b = mp_ref[...] * SCALE
    x1, y1, x2, y2 = b[:, 0:1], b[:, 1:2], b[:, 2:3], b[:, 3:4]
    bin_w = jnp.maximum(x2 - x1, 1.0) / M
    bin_h = jnp.maximum(y2 - y1, 1.0) / M
    g = lax.broadcasted_iota(jnp.int32, (1, M), 1).astype(jnp.float32) + 0.5
    x = jnp.clip(x1 + g * bin_w, 0.0, W - 1.0)
    y = jnp.clip(y1 + g * bin_h, 0.0, H - 1.0)
    x0f = jnp.floor(x)
    y0f = jnp.floor(y)
    x0 = x0f.astype(jnp.int32)
    y0 = y0f.astype(jnp.int32)
    lx = x - x0f
    ly = y - y0f
    hx = 1.0 - lx
    hy = 1.0 - ly
    row0 = y0 * W
    i00 = row0[:, :, None] + x0[:, None, :]
    i10 = i00 + W
    cap = HW - 1
    i00_ref[...] = i00
    i01_ref[...] = jnp.minimum(i00 + 1, cap)
    i10_ref[...] = jnp.minimum(i10, cap)
    i11_ref[...] = jnp.minimum(i10 + 1, cap)
    w00 = hy[:, :, None] * hx[:, None, :]
    w01 = hy[:, :, None] * lx[:, None, :]
    w10 = ly[:, :, None] * hx[:, None, :]
    w11 = ly[:, :, None] * lx[:, None, :]
    wtb_ref[:, :, :, 0:16] = jnp.broadcast_to(w00[..., None], (NB, M, M, 16))
    wtb_ref[:, :, :, 16:32] = jnp.broadcast_to(w01[..., None], (NB, M, M, 16))
    wtb_ref[:, :, :, 32:48] = jnp.broadcast_to(w10[..., None], (NB, M, M, 16))
    wtb_ref[:, :, :, 48:64] = jnp.broadcast_to(w11[..., None], (NB, M, M, 16))


def _sc_gather_body(table, x00, x01, x10, x11, wtb, out,
                    i0a, i1a, i2a, i3a, v0a, v1a, v2a, v3a, wva, oa,
                    i0b, i1b, i2b, i3b, v0b, v1b, v2b, v3b, wvb, ob,
                    sga, sgb, swa, swb):
    wid = lax.axis_index("s") * 2 + lax.axis_index("c")
    base = wid * PPW

    slots = (((i0a, i1a, i2a, i3a), (v0a, v1a, v2a, v3a), wva, oa, sga, swa),
             ((i0b, i1b, i2b, i3b), (v0b, v1b, v2b, v3b), wvb, ob, sgb, swb))
    srcs = (x00, x01, x10, x11)

    def fire(ci, s):
        idx, vbufs, wv_s, _, sg_s, _ = s
        p0 = base + ci * CHUNK
        for j in range(4):
            pltpu.sync_copy(srcs[j].at[pl.ds(p0, CHUNK)], idx[j])
        for j in range(4):
            pltpu.async_copy(table.at[idx[j]], vbufs[j], sg_s)
        pltpu.async_copy(wtb.at[pl.ds(p0, CHUNK)], wv_s, sg_s)

    def drain_g(s):
        idx, vbufs, wv_s, _, sg_s, _ = s
        for j in range(4):
            pltpu.make_async_copy(table.at[idx[j]], vbufs[j], sg_s).wait()
        pltpu.make_async_copy(wtb.at[pl.ds(0, CHUNK)], wv_s, sg_s).wait()

    def fire_w(ci, s):
        _, _, _, o_s, _, sw_s = s
        p0 = base + ci * CHUNK
        pltpu.async_copy(o_s, out.at[pl.ds(p0, CHUNK)], sw_s)

    def drain_w(s):
        _, _, _, o_s, _, sw_s = s
        pltpu.make_async_copy(o_s, out.at[pl.ds(0, CHUNK)], sw_s).wait()

    def combine(s):
        _, (v0, v1, v2, v3), wv_s, o_s, _, _ = s

        def point(p, pc):
            w0 = wv_s[p, pl.ds(0, 16)]
            w1 = wv_s[p, pl.ds(16, 16)]
            w2 = wv_s[p, pl.ds(32, 16)]
            w3 = wv_s[p, pl.ds(48, 16)]
            for k in range(C // 16):
                sl = pl.ds(k * 16, 16)
                r = (w0 * v0[p, sl] + w1 * v1[p, sl]
                     + w2 * v2[p, sl] + w3 * v3[p, sl])
                o_s[p, sl] = r
            return pc

        lax.fori_loop(0, CHUNK, point, 0)

    fire(0, slots[0])
    fire(1, slots[1])

    for ci in range(2):
        s = slots[ci]
        drain_g(s)
        combine(s)
        fire_w(ci, s)
        fire(ci + 2, s)

    g_end = (NCHUNK - 4) // 2 + 1

    def pair(g, carry):
        for b in range(2):
            ci = 2 * g + b
            s = slots[b]
            drain_g(s)
            drain_w(s)
            combine(s)
            fire_w(ci, s)
            fire(ci + 2, s)
        return carry

    lax.fori_loop(1, g_end, pair, 0)

    for ci in range(2 * g_end, NCHUNK):
        s = slots[ci % 2]
        drain_g(s)
        drain_w(s)
        combine(s)
        fire_w(ci, s)
        if ci + 2 < NCHUNK:
            fire(ci + 2, s)

    drain_w(slots[0])
    drain_w(slots[1])


@functools.lru_cache(maxsize=1)
def _sc_gather():
    ibuf = pltpu.VMEM((CHUNK,), jnp.int32)
    vbuf = pltpu.VMEM((CHUNK, C), jnp.float32)
    wbuf = pltpu.VMEM((CHUNK, 64), jnp.float32)
    return pl.kernel(
        _sc_gather_body,
        mesh=plsc.VectorSubcoreMesh(core_axis_name="c", subcore_axis_name="s"),
        out_type=jax.ShapeDtypeStruct((PS, C), jnp.float32),
        scratch_types=[
            ibuf, ibuf, ibuf, ibuf, vbuf, vbuf, vbuf, vbuf, wbuf, vbuf,
            ibuf, ibuf, ibuf, ibuf, vbuf, vbuf, vbuf, vbuf, wbuf, vbuf,
            pltpu.SemaphoreType.DMA,
            pltpu.SemaphoreType.DMA,
            pltpu.SemaphoreType.DMA,
            pltpu.SemaphoreType.DMA,
        ],
    )


def _mm_body(x_ref, wh_ref, bh_ref, w2_ref, b2_ref, wp_ref, bp_ref, o_ref):
    x = x_ref[...]
    h = jnp.maximum(
        jnp.dot(x, wh_ref[...], preferred_element_type=jnp.float32)
        + bh_ref[...], 0.0)
    u = jnp.maximum(
        jnp.dot(h, w2_ref[...], preferred_element_type=jnp.float32)
        + b2_ref[...], 0.0)
    z = jnp.dot(u, wp_ref[...], preferred_element_type=jnp.float32) + bp_ref[...]
    o_ref[...] = jax.nn.sigmoid(jnp.transpose(z))


def _prep_slice(mp_slice):
    ispec = pl.BlockSpec((NB, M, M), lambda i: (i, 0, 0))
    ishape = jax.ShapeDtypeStruct((RS, M, M), jnp.int32)
    x00, x01, x10, x11, wtb = pl.pallas_call(
        _prep_body,
        grid=(RS // NB,),
        in_specs=[pl.BlockSpec((NB, 4), lambda i: (i, 0))],
        out_specs=[ispec, ispec, ispec, ispec,
                   pl.BlockSpec((NB, M, M, 64), lambda i: (i, 0, 0, 0))],
        out_shape=[ishape, ishape, ishape, ishape,
                   jax.ShapeDtypeStruct((RS, M, M, 64), jnp.float32)],
    )(mp_slice)
    return (x00.reshape(PS), x01.reshape(PS), x10.reshape(PS),
            x11.reshape(PS), wtb.reshape(PS, 64))


def _mm_slice(roi, weights):
    whT, bh, w2r, b2, wp4, bp = weights
    return pl.pallas_call(
        _mm_body,
        grid=(MM_STEPS,),
        in_specs=[
            pl.BlockSpec((MM_ROWS, C), lambda i: (i, 0)),
            pl.BlockSpec((C, 128), lambda i: (0, 0)),
            pl.BlockSpec((1, 128), lambda i: (0, 0)),
            pl.BlockSpec((128, 256), lambda i: (0, 0)),
            pl.BlockSpec((1, 256), lambda i: (0, 0)),
            pl.BlockSpec((256, 4), lambda i: (0, 0)),
            pl.BlockSpec((1, 4), lambda i: (0, 0)),
        ],
        out_specs=pl.BlockSpec((4, MM_ROWS), lambda i: (0, i)),
        out_shape=jax.ShapeDtypeStruct((4, PS), jnp.float32),
    )(roi, whT, bh, w2r, b2, wp4, bp)


def kernel(features, mask_proposals, w_head, b_head, w_deconv, b_deconv,
           w_pred, b_pred):
    table = pl.pallas_call(
        _transpose_body,
        grid=(T_STEPS,),
        in_specs=[pl.BlockSpec((C, H // T_STEPS, W), lambda i: (0, i, 0))],
        out_specs=pl.BlockSpec((T_COLS, C), lambda i: (i, 0)),
        out_shape=jax.ShapeDtypeStruct((HW, C), jnp.float32),
    )(features.reshape(C, H, W))

    whT = jnp.transpose(w_head)
    bh = b_head.reshape(1, 128)
    w2r = w_deconv.reshape(128, 4 * 64)
    b2 = jnp.repeat(b_deconv, 4).reshape(1, 256)
    wp4 = (w_pred[0].reshape(64, 1, 1)
           * jnp.eye(4, dtype=w_pred.dtype).reshape(1, 4, 4)).reshape(256, 4)
    bp = jnp.broadcast_to(b_pred[0:1], (1, 4))
    weights = (whT, bh, w2r, b2, wp4, bp)

    sc = _sc_gather()
    outs = []
    for sl in range(NSLICE):
        mp = mask_proposals[sl * RS:(sl + 1) * RS]
        x00, x01, x10, x11, wtb = _prep_slice(mp)
        roi = sc(table, x00, x01, x10, x11, wtb)
        val = _mm_slice(roi, weights)
        out = val.reshape(2, 2, RS, M, M).transpose(2, 3, 0, 4, 1)
        outs.append(out.reshape(RS, 2 * M, 2 * M))
    return jnp.concatenate(outs, axis=0)


# speedup vs baseline: 1.6310x; 1.0015x over previous
import functools

import jax
import jax.numpy as jnp
from jax import lax
from jax.experimental import pallas as pl
from jax.experimental.pallas import tpu as pltpu
from jax.experimental.pallas import tpu_sc as plsc

H, W = 200, 272
C = 256
M = 14
N_ROIS = 512
P = N_ROIS * M * M
HW = H * W
SCALE = 0.25

NSLICE = 2
RS = N_ROIS // NSLICE
PS = P // NSLICE

NW = 32
PPW = PS // NW
CHUNK = 32
NCHUNK = PPW // CHUNK

T_STEPS = 25
T_COLS = HW // T_STEPS

NB = 32
MM_ROWS = 1024
MM_STEPS = PS // MM_ROWS


def _transpose_body(src_ref, dst_ref):
    dst_ref[...] = src_ref[...].reshape(C, T_COLS).T


def _prep_body(mp_ref, i00_ref, i01_ref, i10_ref, i11_ref, wtb_ref):
    b = mp_ref[...] * SCALE
    x1, y1, x2, y2 = b[:, 0:1], b[:, 1:2], b[:, 2:3], b[:, 3:4]
    bin_w = jnp.maximum(x2 - x1, 1.0) / M
    bin_h = jnp.maximum(y2 - y1, 1.0) / M
    g = lax.broadcasted_iota(jnp.int32, (1, M), 1).astype(jnp.float32) + 0.5
    x = jnp.clip(x1 + g * bin_w, 0.0, W - 1.0)
    y = jnp.clip(y1 + g * bin_h, 0.0, H - 1.0)
    x0f = jnp.floor(x)
    y0f = jnp.floor(y)
    x0 = x0f.astype(jnp.int32)
    y0 = y0f.astype(jnp.int32)
    lx = x - x0f
    ly = y - y0f
    hx = 1.0 - lx
    hy = 1.0 - ly
    row0 = y0 * W
    i00 = row0[:, :, None] + x0[:, None, :]
    i10 = i00 + W
    cap = HW - 1
    i00_ref[...] = i00
    i01_ref[...] = jnp.minimum(i00 + 1, cap)
    i10_ref[...] = jnp.minimum(i10, cap)
    i11_ref[...] = jnp.minimum(i10 + 1, cap)
    w00 = hy[:, :, None] * hx[:, None, :]
    w01 = hy[:, :, None] * lx[:, None, :]
    w10 = ly[:, :, None] * hx[:, None, :]
    w11 = ly[:, :, None] * lx[:, None, :]
    wtb_ref[:, :, :, 0:16] = jnp.broadcast_to(w00[..., None], (NB, M, M, 16))
    wtb_ref[:, :, :, 16:32] = jnp.broadcast_to(w01[..., None], (NB, M, M, 16))
    wtb_ref[:, :, :, 32:48] = jnp.broadcast_to(w10[..., None], (NB, M, M, 16))
    wtb_ref[:, :, :, 48:64] = jnp.broadcast_to(w11[..., None], (NB, M, M, 16))


def _sc_gather_body(table, x00, x01, x10, x11, wtb, out,
                    i0a, i1a, i2a, i3a, v0a, v1a, v2a, v3a, wva, oa,
                    i0b, i1b, i2b, i3b, v0b, v1b, v2b, v3b, wvb, ob,
                    sga, sgb, swa, swb):
    wid = lax.axis_index("s") * 2 + lax.axis_index("c")
    base = wid * PPW

    slots = (((i0a, i1a, i2a, i3a), (v0a, v1a, v2a, v3a), wva, oa, sga, swa),
             ((i0b, i1b, i2b, i3b), (v0b, v1b, v2b, v3b), wvb, ob, sgb, swb))
    srcs = (x00, x01, x10, x11)

    def fire(ci, s):
        idx, vbufs, wv_s, _, sg_s, _ = s
        p0 = base + ci * CHUNK
        for j in range(4):
            pltpu.sync_copy(srcs[j].at[pl.ds(p0, CHUNK)], idx[j])
        for j in range(4):
            pltpu.async_copy(table.at[idx[j]], vbufs[j], sg_s)
        pltpu.async_copy(wtb.at[pl.ds(p0, CHUNK)], wv_s, sg_s)

    def drain_g(s):
        idx, vbufs, wv_s, _, sg_s, _ = s
        for j in range(4):
            pltpu.make_async_copy(table.at[idx[j]], vbufs[j], sg_s).wait()
        pltpu.make_async_copy(wtb.at[pl.ds(0, CHUNK)], wv_s, sg_s).wait()

    def fire_w(ci, s):
        _, _, _, o_s, _, sw_s = s
        p0 = base + ci * CHUNK
        pltpu.async_copy(o_s, out.at[pl.ds(p0, CHUNK)], sw_s)

    def drain_w(s):
        _, _, _, o_s, _, sw_s = s
        pltpu.make_async_copy(o_s, out.at[pl.ds(0, CHUNK)], sw_s).wait()

    def combine(s):
        _, (v0, v1, v2, v3), wv_s, o_s, _, _ = s

        def point(p, pc):
            w0 = wv_s[p, pl.ds(0, 16)]
            w1 = wv_s[p, pl.ds(16, 16)]
            w2 = wv_s[p, pl.ds(32, 16)]
            w3 = wv_s[p, pl.ds(48, 16)]
            for k in range(C // 16):
                sl = pl.ds(k * 16, 16)
                r = (w0 * v0[p, sl] + w1 * v1[p, sl]
                     + w2 * v2[p, sl] + w3 * v3[p, sl])
                o_s[p, sl] = r
            return pc

        lax.fori_loop(0, CHUNK, point, 0)

    fire(0, slots[0])
    fire(1, slots[1])

    for ci in range(2):
        s = slots[ci]
        drain_g(s)
        combine(s)
        fire_w(ci, s)
        fire(ci + 2, s)

    g_end = (NCHUNK - 4) // 2 + 1

    def pair(g, carry):
        for b in range(2):
            ci = 2 * g + b
            s = slots[b]
            drain_g(s)
            drain_w(s)
            combine(s)
            fire_w(ci, s)
            fire(ci + 2, s)
        return carry

    lax.fori_loop(1, g_end, pair, 0)

    for ci in range(2 * g_end, NCHUNK):
        s = slots[ci % 2]
        drain_g(s)
        drain_w(s)
        combine(s)
        fire_w(ci, s)
        if ci + 2 < NCHUNK:
            fire(ci + 2, s)

    drain_w(slots[0])
    drain_w(slots[1])


@functools.lru_cache(maxsize=1)
def _sc_gather():
    ibuf = pltpu.VMEM((CHUNK,), jnp.int32)
    vbuf = pltpu.VMEM((CHUNK, C), jnp.float32)
    wbuf = pltpu.VMEM((CHUNK, 64), jnp.float32)
    return pl.kernel(
        _sc_gather_body,
        mesh=plsc.VectorSubcoreMesh(core_axis_name="c", subcore_axis_name="s"),
        out_type=jax.ShapeDtypeStruct((PS, C), jnp.float32),
        scratch_types=[
            ibuf, ibuf, ibuf, ibuf, vbuf, vbuf, vbuf, vbuf, wbuf, vbuf,
            ibuf, ibuf, ibuf, ibuf, vbuf, vbuf, vbuf, vbuf, wbuf, vbuf,
            pltpu.SemaphoreType.DMA,
            pltpu.SemaphoreType.DMA,
            pltpu.SemaphoreType.DMA,
            pltpu.SemaphoreType.DMA,
        ],
    )


def _mm_body(x_ref, wh_ref, bh_ref, w2_ref, b2_ref, wp_ref, bp_ref, o_ref):
    x = x_ref[...].astype(jnp.bfloat16)
    h = jnp.maximum(
        jnp.dot(x, wh_ref[...], preferred_element_type=jnp.float32)
        + bh_ref[...], 0.0)
    u = jnp.maximum(
        jnp.dot(h.astype(jnp.bfloat16), w2_ref[...],
                preferred_element_type=jnp.float32)
        + b2_ref[...], 0.0)
    z = jnp.dot(u, wp_ref[...], preferred_element_type=jnp.float32) + bp_ref[...]
    o_ref[...] = jax.nn.sigmoid(jnp.transpose(z))


def _prep_slice(mp_slice):
    ispec = pl.BlockSpec((NB, M, M), lambda i: (i, 0, 0))
    ishape = jax.ShapeDtypeStruct((RS, M, M), jnp.int32)
    x00, x01, x10, x11, wtb = pl.pallas_call(
        _prep_body,
        grid=(RS // NB,),
        in_specs=[pl.BlockSpec((NB, 4), lambda i: (i, 0))],
        out_specs=[ispec, ispec, ispec, ispec,
                   pl.BlockSpec((NB, M, M, 64), lambda i: (i, 0, 0, 0))],
        out_shape=[ishape, ishape, ishape, ishape,
                   jax.ShapeDtypeStruct((RS, M, M, 64), jnp.float32)],
    )(mp_slice)
    return (x00.reshape(PS), x01.reshape(PS), x10.reshape(PS),
            x11.reshape(PS), wtb.reshape(PS, 64))


def _mm_slice(roi, weights):
    whT, bh, w2r, b2, wp4, bp = weights
    return pl.pallas_call(
        _mm_body,
        grid=(MM_STEPS,),
        in_specs=[
            pl.BlockSpec((MM_ROWS, C), lambda i: (i, 0)),
            pl.BlockSpec((C, 128), lambda i: (0, 0)),
            pl.BlockSpec((1, 128), lambda i: (0, 0)),
            pl.BlockSpec((128, 256), lambda i: (0, 0)),
            pl.BlockSpec((1, 256), lambda i: (0, 0)),
            pl.BlockSpec((256, 4), lambda i: (0, 0)),
            pl.BlockSpec((1, 4), lambda i: (0, 0)),
        ],
        out_specs=pl.BlockSpec((4, MM_ROWS), lambda i: (0, i)),
        out_shape=jax.ShapeDtypeStruct((4, PS), jnp.float32),
    )(roi, whT, bh, w2r, b2, wp4, bp)


def kernel(features, mask_proposals, w_head, b_head, w_deconv, b_deconv,
           w_pred, b_pred):
    table = pl.pallas_call(
        _transpose_body,
        grid=(T_STEPS,),
        in_specs=[pl.BlockSpec((C, H // T_STEPS, W), lambda i: (0, i, 0))],
        out_specs=pl.BlockSpec((T_COLS, C), lambda i: (i, 0)),
        out_shape=jax.ShapeDtypeStruct((HW, C), jnp.float32),
    )(features.reshape(C, H, W))

    whT = jnp.transpose(w_head).astype(jnp.bfloat16)
    bh = b_head.reshape(1, 128)
    w2r = w_deconv.reshape(128, 4 * 64).astype(jnp.bfloat16)
    b2 = jnp.repeat(b_deconv, 4).reshape(1, 256)
    wp4 = (w_pred[0].reshape(64, 1, 1)
           * jnp.eye(4, dtype=w_pred.dtype).reshape(1, 4, 4)).reshape(256, 4)
    bp = jnp.broadcast_to(b_pred[0:1], (1, 4))
    weights = (whT, bh, w2r, b2, wp4, bp)

    sc = _sc_gather()
    outs = []
    for sl in range(NSLICE):
        mp = mask_proposals[sl * RS:(sl + 1) * RS]
        x00, x01, x10, x11, wtb = _prep_slice(mp)
        roi = sc(table, x00, x01, x10, x11, wtb)
        val = _mm_slice(roi, weights)
        out = val.reshape(2, 2, RS, M, M).transpose(2, 3, 0, 4, 1)
        outs.append(out.reshape(RS, 2 * M, 2 * M))
    return jnp.concatenate(outs, axis=0)
